# baseline (speedup 1.0000x reference)
import functools

import jax
import jax.numpy as jnp
from jax import lax
from jax.experimental import pallas as pl
from jax.experimental.pallas import tpu as pltpu
from jax.experimental.pallas import tpu_sc as plsc

_BATCH = 16384
_FIELDS = 26
_DIM = 64
_GROUPS = 7812
_PAIR_ROWS = 500008
_TAIL_PAIR0 = 499968

_mesh = plsc.VectorSubcoreMesh(core_axis_name="c", subcore_axis_name="s")


def _wid():
    return lax.axis_index("s") * 2 + lax.axis_index("c")


@functools.partial(
    pl.kernel,
    out_type=jax.ShapeDtypeStruct((_PAIR_ROWS, 128), jnp.float32),
    mesh=_mesh,
    scratch_types=[
        pltpu.VMEM((2, 64, 129), jnp.float32),
        pltpu.VMEM((2, 64, 128), jnp.float32),
        pltpu.SemaphoreType.DMA,
        pltpu.SemaphoreType.DMA,
        pltpu.SemaphoreType.DMA,
        pltpu.SemaphoreType.DMA,
    ],
    compiler_params=pltpu.CompilerParams(needs_layout_passes=False),
)
def _convert_kernel(tbl_t, tail2, scratch, sbuf, dbuf, gi0, gi1, go0, go1):
    gin = (gi0, gi1)
    gout = (go0, go1)
    w = _wid()
    n_g = 244 + jnp.where(w < 4, 1, 0)
    g0 = w * 244 + jnp.minimum(w, 4)

    iota = lax.iota(jnp.int32, 16)
    zeros = jnp.zeros((16,), jnp.int32)
    iota2 = iota * 2
    rots = [(iota + s) % 16 for s in range(16)]

    @pl.when(w == 0)
    def _tail():
        pltpu.sync_copy(tail2, dbuf.at[0].at[pl.ds(0, 32)])
        pltpu.sync_copy(dbuf.at[0].at[pl.ds(0, 32)],
                        scratch.at[pl.ds(_TAIL_PAIR0, 32)])

    def load(c, b):
        return pltpu.make_async_copy(
            tbl_t.at[:, pl.ds(c * 128, 128)],
            sbuf.at[b].at[:, pl.ds(0, 128)], gin[b])

    def store(c, b):
        return pltpu.make_async_copy(
            dbuf.at[b], scratch.at[pl.ds(c * 64, 64)], gout[b])

    def transpose(b):
        @plsc.parallel_loop(0, 16, unroll=4)
        def per_rot(s):
            rot = rots[0] + s
            rot = rot - (rot >= 16).astype(jnp.int32) * 16
            for jb in range(4):
                jv = zeros + jb * 16
                for xc in range(8):
                    rv = (zeros + (xc % 4) * 16) + rot
                    cv = iota2 + (jb * 32 + xc // 4)
                    v = plsc.load_gather(sbuf.at[b], [rv, cv])
                    plsc.store_scatter(
                        dbuf.at[b], [jv + iota, (zeros + xc * 16) + rot], v)

    load(g0, 0).start()
    load(g0 + 1, 1).start()

    def pipe(i2, carry):
        for b in range(2):
            i = i2 * 2 + b
            c = g0 + i
            load(c, b).wait()

            @pl.when(i >= 2)
            def _drain_prev():
                store(c - 2, b).wait()

            transpose(b)
            store(c, b).start()

            @pl.when(i + 2 < 244)
            def _start_next():
                load(c + 2, b).start()
        return carry

    lax.fori_loop(0, 122, pipe, 0)

    store(g0 + 242, 0).wait()
    store(g0 + 243, 1).wait()

    @pl.when(n_g > 244)
    def _extra():
        c = g0 + 244
        load(c, 0).start()
        load(c, 0).wait()
        transpose(0)
        store(c, 0).start()
        store(c, 0).wait()


@functools.partial(
    pl.kernel,
    out_type=jax.ShapeDtypeStruct((_FIELDS, 8, 128, 8, 128), jnp.float32),
    mesh=_mesh,
    scratch_types=[
        pltpu.VMEM((128, _FIELDS), jnp.int32),
        pltpu.VMEM((32, 128), jnp.int32),
        pltpu.VMEM((32, 128), jnp.int32),
        pltpu.VMEM((2, 128, 128), jnp.float32),
        pltpu.VMEM((2, 8, 8, 128), jnp.float32),
        pltpu.SemaphoreType.DMA,
        pltpu.SemaphoreType.DMA,
        pltpu.SemaphoreType.DMA,
        pltpu.SemaphoreType.DMA,
        pltpu.SemaphoreType.DMA,
    ],
    compiler_params=pltpu.CompilerParams(
        use_tc_tiling_on_sc=False, needs_layout_passes=False),
)
def _gather_kernel(keys_hbm, scratch, out_hbm, kbuf, krow, khalf,
                   gbuf, pbuf, ks, gi0, gi1, go0, go1):
    gin = (gi0, gi1)
    gout = (go0, go1)
    w = _wid()
    iota = lax.iota(jnp.int32, 16)
    zeros = jnp.zeros((16,), jnp.int32)

    def stage_keys(cb):
        pltpu.sync_copy(keys_hbm.at[pl.ds(cb * 128, 128)], kbuf)

        @plsc.parallel_loop(0, _FIELDS, unroll=2)
        def per_field(f):
            fv = zeros + f
            for rc in range(8):
                rv = iota + rc * 16
                v = plsc.load_gather(kbuf, [rv, fv])
                krow[f, pl.ds(rc * 16, 16)] = lax.shift_right_logical(v, 1)
                khalf[f, pl.ds(rc * 16, 16)] = lax.shift_left(
                    lax.bitwise_and(v, 1), 6)

    def gather(f, b):
        return pltpu.make_async_copy(
            scratch.at[krow.at[f]], gbuf.at[b], gin[b])

    def out_dma(f, cb, b):
        return pltpu.make_async_copy(
            pbuf.at[b], out_hbm.at[f, :, cb, :, :], gout[b])

    def transpose(f, b):
        halfs = [khalf[f, pl.ds(lb * 16, 16)] for lb in range(8)]

        @plsc.parallel_loop(0, 16, unroll=2)
        def per_rot(s):
            rot = iota + s
            rot = rot - (rot >= 16).astype(jnp.int32) * 16
            for db in range(4):
                d_vec = rot + db * 16
                t_vec = lax.shift_right_logical(d_vec, 3)
                s_vec = lax.bitwise_and(d_vec, 7)
                for lb in range(8):
                    v = plsc.load_gather(
                        gbuf.at[b], [iota + lb * 16, halfs[lb] + d_vec])
                    plsc.store_scatter(
                        pbuf.at[b], [t_vec, s_vec, iota + lb * 16], v)

    def per_block(cb_i, carry):
        cb = w * 4 + cb_i
        stage_keys(cb)
        gather(0, 0).start()
        gather(1, 1).start()

        def per_field(fp, carry2):
            for b in range(2):
                f = fp * 2 + b
                gather(f, b).wait()

                @pl.when(f >= 2)
                def _drain_prev():
                    out_dma(f - 2, cb, b).wait()

                transpose(f, b)
                out_dma(f, cb, b).start()

                @pl.when(f + 2 < _FIELDS)
                def _start_next():
                    gather(f + 2, b).start()
            return carry2

        lax.fori_loop(0, _FIELDS // 2, per_field, 0)
        out_dma(_FIELDS - 2, cb, 0).wait()
        out_dma(_FIELDS - 1, cb, 1).wait()
        return carry

    lax.fori_loop(0, 4, per_block, 0)


def kernel(keys, table):
    tbl_t = table.T
    tail2 = table[999936:1000000].reshape(32, 128)
    scratch = _convert_kernel(tbl_t, tail2)
    out5 = _gather_kernel(keys, scratch)
    return out5.transpose((2, 4, 0, 1, 3)).reshape(_BATCH, _FIELDS, _DIM)


# speedup vs baseline: 1.0140x; 1.0140x over previous
import functools

import jax
import jax.numpy as jnp
from jax import lax
from jax.experimental import pallas as pl
from jax.experimental.pallas import tpu as pltpu
from jax.experimental.pallas import tpu_sc as plsc

_BATCH = 16384
_FIELDS = 26
_DIM = 64
_GROUPS = 7812
_PAIR_ROWS = 500008
_TAIL_PAIR0 = 499968

_mesh = plsc.VectorSubcoreMesh(core_axis_name="c", subcore_axis_name="s")


def _wid():
    return lax.axis_index("s") * 2 + lax.axis_index("c")


@functools.partial(
    pl.kernel,
    out_type=jax.ShapeDtypeStruct((_PAIR_ROWS, 128), jnp.float32),
    mesh=_mesh,
    scratch_types=[
        pltpu.VMEM((2, 64, 128), jnp.float32),
        pltpu.VMEM((2, 64, 128), jnp.float32),
        pltpu.SemaphoreType.DMA,
        pltpu.SemaphoreType.DMA,
        pltpu.SemaphoreType.DMA,
        pltpu.SemaphoreType.DMA,
    ],
    compiler_params=pltpu.CompilerParams(needs_layout_passes=False),
)
def _convert_kernel(tbl_t, tail2, scratch, sbuf, dbuf, gi0, gi1, go0, go1):
    gin = (gi0, gi1)
    gout = (go0, go1)
    w = _wid()
    n_g = 244 + jnp.where(w < 4, 1, 0)
    g0 = w * 244 + jnp.minimum(w, 4)

    iota = lax.iota(jnp.int32, 16)
    zeros = jnp.zeros((16,), jnp.int32)
    iota2 = iota * 2
    rots = [(iota + s) % 16 for s in range(16)]

    @pl.when(w == 0)
    def _tail():
        pltpu.sync_copy(tail2, sbuf.at[0].at[pl.ds(0, 32)])
        pltpu.sync_copy(sbuf.at[0].at[pl.ds(0, 32)],
                        scratch.at[pl.ds(_TAIL_PAIR0, 32)])

    def load(c, b):
        return pltpu.make_async_copy(
            tbl_t.at[:, pl.ds(c * 128, 128)], sbuf.at[b], gin[b])

    def store(c, b):
        return pltpu.make_async_copy(
            dbuf.at[b], scratch.at[pl.ds(c * 64, 64)], gout[b])

    def transpose(b):
        @plsc.parallel_loop(0, 16, unroll=4)
        def per_rot(s):
            rot = rots[0] + s
            rot = rot - (rot >= 16).astype(jnp.int32) * 16
            for jb in range(4):
                jv = zeros + jb * 16
                for xc in range(8):
                    rv = (zeros + (xc % 4) * 16) + rot
                    cv = iota2 + (jb * 32 + xc // 4)
                    v = plsc.load_gather(sbuf.at[b], [rv, cv])
                    plsc.store_scatter(
                        dbuf.at[b], [jv + iota, (zeros + xc * 16) + rot], v)

    load(g0, 0).start()
    load(g0 + 1, 1).start()

    def pipe(i2, carry):
        for b in range(2):
            i = i2 * 2 + b
            c = g0 + i
            load(c, b).wait()

            @pl.when(i >= 2)
            def _drain_prev():
                store(c - 2, b).wait()

            transpose(b)
            store(c, b).start()

            @pl.when(i + 2 < 244)
            def _start_next():
                load(c + 2, b).start()
        return carry

    lax.fori_loop(0, 122, pipe, 0)

    store(g0 + 242, 0).wait()
    store(g0 + 243, 1).wait()

    @pl.when(n_g > 244)
    def _extra():
        c = g0 + 244
        load(c, 0).start()
        load(c, 0).wait()
        transpose(0)
        store(c, 0).start()
        store(c, 0).wait()


@functools.partial(
    pl.kernel,
    out_type=jax.ShapeDtypeStruct((_FIELDS, 8, 128, 8, 128), jnp.float32),
    mesh=_mesh,
    scratch_types=[
        pltpu.VMEM((128, _FIELDS), jnp.int32),
        pltpu.VMEM((32, 128), jnp.int32),
        pltpu.VMEM((32, 128), jnp.int32),
        pltpu.VMEM((2, 128, 128), jnp.float32),
        pltpu.VMEM((2, 8, 8, 128), jnp.float32),
        pltpu.SemaphoreType.DMA,
        pltpu.SemaphoreType.DMA,
        pltpu.SemaphoreType.DMA,
        pltpu.SemaphoreType.DMA,
        pltpu.SemaphoreType.DMA,
    ],
    compiler_params=pltpu.CompilerParams(
        use_tc_tiling_on_sc=False, needs_layout_passes=False),
)
def _gather_kernel(keys_hbm, scratch, out_hbm, kbuf, krow, khalf,
                   gbuf, pbuf, ks, gi0, gi1, go0, go1):
    gin = (gi0, gi1)
    gout = (go0, go1)
    w = _wid()
    iota = lax.iota(jnp.int32, 16)
    zeros = jnp.zeros((16,), jnp.int32)

    def stage_keys(cb):
        pltpu.sync_copy(keys_hbm.at[pl.ds(cb * 128, 128)], kbuf)

        @plsc.parallel_loop(0, _FIELDS, unroll=2)
        def per_field(f):
            fv = zeros + f
            for rc in range(8):
                rv = iota + rc * 16
                v = plsc.load_gather(kbuf, [rv, fv])
                krow[f, pl.ds(rc * 16, 16)] = lax.shift_right_logical(v, 1)
                khalf[f, pl.ds(rc * 16, 16)] = lax.shift_left(
                    lax.bitwise_and(v, 1), 6)

    def gather(f, b):
        return pltpu.make_async_copy(
            scratch.at[krow.at[f]], gbuf.at[b], gin[b])

    def out_dma(f, cb, b):
        return pltpu.make_async_copy(
            pbuf.at[b], out_hbm.at[f, :, cb, :, :], gout[b])

    def transpose(f, b):
        halfs = [khalf[f, pl.ds(lb * 16, 16)] for lb in range(8)]

        @plsc.parallel_loop(0, 16, unroll=2)
        def per_rot(s):
            rot = iota + s
            rot = rot - (rot >= 16).astype(jnp.int32) * 16
            for db in range(4):
                d_vec = rot + db * 16
                t_vec = lax.shift_right_logical(d_vec, 3)
                s_vec = lax.bitwise_and(d_vec, 7)
                for lb in range(8):
                    v = plsc.load_gather(
                        gbuf.at[b], [iota + lb * 16, halfs[lb] + d_vec])
                    plsc.store_scatter(
                        pbuf.at[b], [t_vec, s_vec, iota + lb * 16], v)

    def per_block(cb_i, carry):
        cb = w * 4 + cb_i
        stage_keys(cb)
        gather(0, 0).start()
        gather(1, 1).start()

        def per_field(fp, carry2):
            for b in range(2):
                f = fp * 2 + b
                gather(f, b).wait()

                @pl.when(f >= 2)
                def _drain_prev():
                    out_dma(f - 2, cb, b).wait()

                transpose(f, b)
                out_dma(f, cb, b).start()

                @pl.when(f + 2 < _FIELDS)
                def _start_next():
                    gather(f + 2, b).start()
            return carry2

        lax.fori_loop(0, _FIELDS // 2, per_field, 0)
        out_dma(_FIELDS - 2, cb, 0).wait()
        out_dma(_FIELDS - 1, cb, 1).wait()
        return carry

    lax.fori_loop(0, 4, per_block, 0)


def kernel(keys, table):
    tbl_t = table.T
    tail2 = table[999936:1000000].reshape(32, 128)
    scratch = _convert_kernel(tbl_t, tail2)
    out5 = _gather_kernel(keys, scratch)
    return out5.transpose((2, 4, 0, 1, 3)).reshape(_BATCH, _FIELDS, _DIM)
